# baseline (speedup 1.0000x reference)
import jax
import jax.numpy as jnp
from jax import lax
from jax.experimental import pallas as pl
from jax.experimental.pallas import tpu as pltpu
from jax.experimental.pallas import tpu_sc as plsc

N = 10000
E = 160000
F_IN = 128
HC = 256
HQ = 64
AW = 80
G = 64
HID = 64
EMB = 256
NC, NS, L = 2, 16, 16

EC = 80
E_TILE = E // NS
N_EC = E_TILE // EC
GRP = EC // L
ROWS_TILE = N // NS
NG = N // L
DIV_ITERS = -(-NG // NS)

RB = 1000
RC = 200
N_RC = N // RC
POOL_ITERS = -(-N_RC // NS)


def _sc_mesh():
    return plsc.VectorSubcoreMesh(
        core_axis_name="c", subcore_axis_name="s",
        num_cores=NC, num_subcores=NS)


_SC_PARAMS = pltpu.CompilerParams(needs_layout_passes=False,
                                  use_tc_tiling_on_sc=False,
                                  disable_bounds_checks=True)


def _conv_body(call, xlh, xrh, src_h, dst_h, attb_h, out_h,
               xl0_v, xr0_v, xl1_v, xr1_v, val0_v, val1_v, attb_v,
               obuf_v, dbuf_v, srcs_v, dsts_v,
               acc_s, g0a, g0b, g1a, g1b):
    c = lax.axis_index("c")
    s = lax.axis_index("s")
    head = 2 * call + c
    iota = lax.iota(jnp.int32, L)
    zf = jnp.zeros((L,), jnp.float32)

    pltpu.sync_copy(attb_h.at[head], attb_v)
    pltpu.sync_copy(src_h.at[s], srcs_v)
    pltpu.sync_copy(dst_h.at[s], dsts_v)
    att4 = [attb_v[pl.ds(q * L, L)] for q in range(HQ // L)]

    def _zero_val(r, carry):
        for q in range(AW // L):
            val0_v[r, pl.ds(q * L, L)] = zf
            val1_v[r, pl.ds(q * L, L)] = zf
        return carry
    lax.fori_loop(0, EC, _zero_val, 0)

    row0 = s * ROWS_TILE

    def _zero_acc(k, carry):
        pltpu.sync_copy(val0_v, acc_s.at[pl.ds(row0 + k * EC, EC)])
        return carry
    lax.fori_loop(0, ROWS_TILE // EC, _zero_acc, 0)
    rem = ROWS_TILE % EC
    pltpu.sync_copy(val0_v.at[pl.ds(0, rem)],
                    acc_s.at[pl.ds(row0 + ROWS_TILE - rem, rem)])

    plsc.subcore_barrier()

    def _issue(k, xl_v, xr_v, sa, sb):
        da = pltpu.async_copy(xlh.at[head].at[srcs_v.at[k]], xl_v, sa)
        db = pltpu.async_copy(xrh.at[head].at[dsts_v.at[k]], xr_v, sb)
        return da, db

    def _compute(k, xl_v, xr_v, val_v):
        @plsc.parallel_loop(0, GRP)
        def _group(g):
            base = g * L
            av = zf
            for j in range(L):
                e = base + j
                dot = zf
                for q in range(HQ // L):
                    a = xl_v[e, pl.ds(q * L, L)]
                    b = xr_v[e, pl.ds(q * L, L)]
                    t = a + b
                    m = jnp.maximum(t, t * 0.2)
                    dot = dot + m * att4[q]
                av = jnp.where(iota == j, jnp.sum(dot), av)
            ae = jnp.exp(av)
            plsc.store_scatter(val_v, [base + iota,
                                       jnp.full((L,), HQ, jnp.int32)], ae)
            for j in range(L):
                e = base + j
                aej = ae[j]
                for q in range(HQ // L):
                    val_v[e, pl.ds(q * L, L)] = xl_v[e, pl.ds(q * L, L)] * aej
        pltpu.sync_copy(val_v, acc_s.at[dsts_v.at[k]], add=True)

    _issue(0, xl0_v, xr0_v, g0a, g0b)

    def _pair(t, carry):
        k0 = 2 * t
        k1 = 2 * t + 1
        _issue(k1, xl1_v, xr1_v, g1a, g1b)
        pltpu.make_async_copy(xlh.at[head].at[srcs_v.at[k0]], xl0_v, g0a).wait()
        pltpu.make_async_copy(xrh.at[head].at[dsts_v.at[k0]], xr0_v, g0b).wait()
        _compute(k0, xl0_v, xr0_v, val0_v)
        _issue(k1 + 1, xl0_v, xr0_v, g0a, g0b)
        pltpu.make_async_copy(xlh.at[head].at[srcs_v.at[k1]], xl1_v, g1a).wait()
        pltpu.make_async_copy(xrh.at[head].at[dsts_v.at[k1]], xr1_v, g1b).wait()
        _compute(k1, xl1_v, xr1_v, val1_v)
        return carry
    lax.fori_loop(0, (N_EC - 1) // 2, _pair, 0)

    kt = N_EC - 1
    pltpu.make_async_copy(xlh.at[head].at[srcs_v.at[kt]], xl0_v, g0a).wait()
    pltpu.make_async_copy(xrh.at[head].at[dsts_v.at[kt]], xr0_v, g0b).wait()
    _compute(kt, xl0_v, xr0_v, val0_v)

    plsc.subcore_barrier()

    def _div(t, carry):
        gi = s + t * NS

        @pl.when(gi < NG)
        def _():
            rb = gi * L
            pltpu.sync_copy(acc_s.at[pl.ds(rb, L)], dbuf_v)
            for j in range(L):
                dv = dbuf_v[j, pl.ds(HQ, L)]
                inv = (1.0 / (dv + 1e-16))[0]
                for q in range(HQ // L):
                    obuf_v[j, pl.ds(q * L, L)] = (
                        dbuf_v[j, pl.ds(q * L, L)] * inv)
            pltpu.sync_copy(obuf_v, out_h.at[c, pl.ds(rb, L)])
        return carry
    lax.fori_loop(0, DIV_ITERS, _div, 0)


def _conv_heads(call, xlq, xrq, src, dst, attb):
    def body(*refs):
        _conv_body(call, *refs)
    k = pl.kernel(
        body,
        out_type=jax.ShapeDtypeStruct((NC, N, HQ), jnp.float32),
        mesh=_sc_mesh(), compiler_params=_SC_PARAMS,
        scratch_types=[
            pltpu.VMEM((EC, HQ), jnp.float32),
            pltpu.VMEM((EC, HQ), jnp.float32),
            pltpu.VMEM((EC, HQ), jnp.float32),
            pltpu.VMEM((EC, HQ), jnp.float32),
            pltpu.VMEM((EC, AW), jnp.float32),
            pltpu.VMEM((EC, AW), jnp.float32),
            pltpu.VMEM((HQ,), jnp.float32),
            pltpu.VMEM((L, HQ), jnp.float32),
            pltpu.VMEM((L, AW), jnp.float32),
            pltpu.VMEM((N_EC, EC), jnp.int32),
            pltpu.VMEM((N_EC, EC), jnp.int32),
            pltpu.VMEM_SHARED((N, AW), jnp.float32),
            pltpu.SemaphoreType.DMA,
            pltpu.SemaphoreType.DMA,
            pltpu.SemaphoreType.DMA,
            pltpu.SemaphoreType.DMA,
        ])
    return k(xlq, xrq, src, dst, attb)


def _pool_body(h_h, batch_h, psum_h, pcnt_h,
               rows_v, ones_v, zb_v, bidx_v, acc_s, cnt_s):
    c = lax.axis_index("c")
    s = lax.axis_index("s")
    onesf = jnp.ones((L,), jnp.float32)
    zf = jnp.zeros((L,), jnp.float32)

    def _fill_ones(r, carry):
        ones_v[r, :] = onesf
        return carry
    lax.fori_loop(0, RC, _fill_ones, 0)

    def _fill_z(r, carry):
        zb_v[r, :] = zf
        for q in range(128 // L):
            rows_v[r, pl.ds(q * L, L)] = zf
        return carry
    lax.fori_loop(0, G, _fill_z, 0)

    @pl.when(s == 0)
    def _():
        pltpu.sync_copy(rows_v.at[pl.ds(0, G)], acc_s)
        pltpu.sync_copy(zb_v, cnt_s)

    plsc.subcore_barrier()

    def _chunk(k, carry):
        cid = s + NS * k

        @pl.when(cid < N_RC)
        def _():
            base = cid * RC
            pltpu.sync_copy(h_h.at[c].at[pl.ds(base, RC)], rows_v)
            pltpu.sync_copy(batch_h.at[pl.ds(base, RC)], bidx_v)
            pltpu.sync_copy(rows_v, acc_s.at[bidx_v], add=True)
            pltpu.sync_copy(ones_v, cnt_s.at[bidx_v], add=True)
        return carry
    lax.fori_loop(0, POOL_ITERS, _chunk, 0)

    plsc.subcore_barrier()

    @pl.when(s == 0)
    def _():
        pltpu.sync_copy(acc_s, psum_h.at[c])
        pltpu.sync_copy(cnt_s, pcnt_h.at[c])


def _pool(hnh, batch):
    k = pl.kernel(
        _pool_body,
        out_type=(jax.ShapeDtypeStruct((NC, G, 128), jnp.float32),
                  jax.ShapeDtypeStruct((NC, G, L), jnp.float32)),
        mesh=_sc_mesh(), compiler_params=_SC_PARAMS,
        scratch_types=[
            pltpu.VMEM((RC, 128), jnp.float32),
            pltpu.VMEM((RC, L), jnp.float32),
            pltpu.VMEM((G, L), jnp.float32),
            pltpu.VMEM((RC,), jnp.int32),
            pltpu.VMEM_SHARED((G, 128), jnp.float32),
            pltpu.VMEM_SHARED((G, L), jnp.float32),
        ])
    return k(hnh, batch)


def _q_specs(grid_rank):
    if grid_rank == 1:
        maps = [lambda i: (0, i, 0), lambda i: (1, i, 0)]
    else:
        maps = [lambda i, j: (0, i, 0), lambda i, j: (1, i, 0)]
    return [pl.BlockSpec((1, RB, HQ), m) for m in maps for _ in (0,)]


def _proj_in(x, Wl, Wr):
    def body(x_ref, wl_ref, wr_ref, xl_ref, xr_ref):
        h = x_ref[...]
        rl = jnp.dot(h, wl_ref[...], preferred_element_type=jnp.float32)
        rr = jnp.dot(h, wr_ref[...], preferred_element_type=jnp.float32)
        for q in range(4):
            xl_ref[q] = rl[:, q * HQ:(q + 1) * HQ]
            xr_ref[q] = rr[:, q * HQ:(q + 1) * HQ]
    return pl.pallas_call(
        body,
        grid=(N // RB,),
        in_specs=[pl.BlockSpec((RB, F_IN), lambda i: (i, 0)),
                  pl.BlockSpec((F_IN, HC), lambda i: (0, 0)),
                  pl.BlockSpec((F_IN, HC), lambda i: (0, 0))],
        out_specs=[pl.BlockSpec((4, RB, HQ), lambda i: (0, i, 0)),
                   pl.BlockSpec((4, RB, HQ), lambda i: (0, i, 0))],
        out_shape=[jax.ShapeDtypeStruct((4, N, HQ), jnp.float32)] * 2,
    )(x, Wl, Wr)


def _quarter_in_specs(grid2):
    if grid2:
        return [pl.BlockSpec((1, RB, HQ), lambda i, j: (0, i, 0)),
                pl.BlockSpec((1, RB, HQ), lambda i, j: (1, i, 0)),
                pl.BlockSpec((1, RB, HQ), lambda i, j: (0, i, 0)),
                pl.BlockSpec((1, RB, HQ), lambda i, j: (1, i, 0))]
    return [pl.BlockSpec((1, RB, HQ), lambda i: (0, i, 0)),
            pl.BlockSpec((1, RB, HQ), lambda i: (1, i, 0)),
            pl.BlockSpec((1, RB, HQ), lambda i: (0, i, 0)),
            pl.BlockSpec((1, RB, HQ), lambda i: (1, i, 0))]


def _cat4(a0, a1, b0, b1):
    return jnp.concatenate([a0[0], a1[0], b0[0], b1[0]], axis=1)


def _stats(h01, h23):
    def body(a0, a1, b0, b1, o_ref):
        i = pl.program_id(0)
        hb = _cat4(a0, a1, b0, b1)
        blk = jnp.concatenate(
            [jnp.sum(hb, 0)[None], jnp.sum(hb * hb, 0)[None],
             jnp.zeros((6, HC), jnp.float32)], 0)

        @pl.when(i == 0)
        def _():
            o_ref[...] = jnp.zeros_like(o_ref)
        o_ref[...] += blk
    return pl.pallas_call(
        body,
        grid=(N // RB,),
        in_specs=_quarter_in_specs(False),
        out_specs=pl.BlockSpec((8, HC), lambda i: (0, 0)),
        out_shape=jax.ShapeDtypeStruct((8, HC), jnp.float32),
    )(h01, h01, h23, h23)


def _bn_from_stats(hb, st, g, be):
    mu = st[0] / N
    var = st[1] / N - mu * mu
    xn = g * (hb - mu) * lax.rsqrt(var + 1e-5) + be
    return jnp.maximum(xn, xn * 0.01)


def _bn_act_proj(h01, h23, st, g2d, be2d, Wl, Wr):
    def body(a0, a1, b0, b1, st_ref, g_ref, be_ref, wl_ref, wr_ref,
             xl_ref, xr_ref):
        hb = _cat4(a0, a1, b0, b1)
        hn = _bn_from_stats(hb, st_ref[...], g_ref[0], be_ref[0])
        rl = jnp.dot(hn, wl_ref[...], preferred_element_type=jnp.float32)
        rr = jnp.dot(hn, wr_ref[...], preferred_element_type=jnp.float32)
        for q in range(4):
            xl_ref[q] = rl[:, q * HQ:(q + 1) * HQ]
            xr_ref[q] = rr[:, q * HQ:(q + 1) * HQ]
    return pl.pallas_call(
        body,
        grid=(N // RB,),
        in_specs=_quarter_in_specs(False) + [
            pl.BlockSpec((8, HC), lambda i: (0, 0)),
            pl.BlockSpec((1, HC), lambda i: (0, 0)),
            pl.BlockSpec((1, HC), lambda i: (0, 0)),
            pl.BlockSpec((HC, HC), lambda i: (0, 0)),
            pl.BlockSpec((HC, HC), lambda i: (0, 0))],
        out_specs=[pl.BlockSpec((4, RB, HQ), lambda i: (0, i, 0)),
                   pl.BlockSpec((4, RB, HQ), lambda i: (0, i, 0))],
        out_shape=[jax.ShapeDtypeStruct((4, N, HQ), jnp.float32)] * 2,
    )(h01, h01, h23, h23, st, g2d, be2d, Wl, Wr)


def _bn_act_halves(h01, h23, st, g2d, be2d):
    def body(a0, a1, b0, b1, st_ref, g_ref, be_ref, o_ref):
        hb = _cat4(a0, a1, b0, b1)
        hn = _bn_from_stats(hb, st_ref[...], g_ref[0], be_ref[0])
        o_ref[0] = hn[:, :128]
        o_ref[1] = hn[:, 128:]
    return pl.pallas_call(
        body,
        grid=(N // RB,),
        in_specs=_quarter_in_specs(False) + [
            pl.BlockSpec((8, HC), lambda i: (0, 0)),
            pl.BlockSpec((1, HC), lambda i: (0, 0)),
            pl.BlockSpec((1, HC), lambda i: (0, 0))],
        out_specs=pl.BlockSpec((2, RB, 128), lambda i: (0, i, 0)),
        out_shape=jax.ShapeDtypeStruct((2, N, 128), jnp.float32),
    )(h01, h01, h23, h23, st, g2d, be2d)


def _head(psum, pcnt, fc1_w, gfc, bfc, fc2_w, gemb, bemb):
    def body(ps_ref, pc_ref, w1_ref, g1_ref, b1_ref, w2_ref, g2_ref, b2_ref,
             o_ref):
        ps = ps_ref[...]
        pc = pc_ref[...]
        sums = jnp.concatenate([ps[0], ps[1]], axis=1)
        cnt = pc[0, :, 0:1]
        hg = sums / jnp.maximum(cnt, 1.0)
        z = jnp.dot(hg, w1_ref[...], preferred_element_type=jnp.float32)
        mu = jnp.mean(z, 0)
        var = jnp.mean(z * z, 0) - mu * mu
        z = g1_ref[0] * (z - mu) * lax.rsqrt(var + 1e-5) + b1_ref[0]
        z = jnp.maximum(z, z * 0.01)
        z2 = jnp.dot(z, w2_ref[...], preferred_element_type=jnp.float32)
        mu2 = jnp.mean(z2, 0)
        var2 = jnp.mean(z2 * z2, 0) - mu2 * mu2
        z2 = g2_ref[0] * (z2 - mu2) * lax.rsqrt(var2 + 1e-5) + b2_ref[0]
        nrm = jnp.sqrt(jnp.sum(z2 * z2, 1, keepdims=True))
        o_ref[...] = z2 / jnp.maximum(nrm, 1e-12)
    return pl.pallas_call(
        body,
        out_shape=jax.ShapeDtypeStruct((G, EMB), jnp.float32),
    )(psum, pcnt, fc1_w, gfc, bfc, fc2_w, gemb, bemb)


def _attb(att):
    return att


def _conv(xlq, xrq, src3, dst3, attb):
    h01 = _conv_heads(0, xlq, xrq, src3, dst3, attb)
    h23 = _conv_heads(1, xlq, xrq, src3, dst3, attb)
    return h01, h23


def kernel(x, edge_index, batch,
           Wl0, Wr0, att0, b0, g0, be0,
           Wl1, Wr1, att1, b1, g1, be1,
           Wl2, Wr2, att2, b2, g2, be2,
           fc1_w, fc1_b, gfc, bfc, fc2_w, fc2_b, gemb, bemb):
    src = edge_index[0].reshape(NS, N_EC, EC)
    dst = edge_index[1].reshape(NS, N_EC, EC)
    xl, xr = _proj_in(x, Wl0, Wr0)
    h01, h23 = _conv(xl, xr, src, dst, _attb(att0))
    st = _stats(h01, h23)
    xl, xr = _bn_act_proj(h01, h23, st, g0.reshape(1, HC), be0.reshape(1, HC),
                          Wl1, Wr1)
    h01, h23 = _conv(xl, xr, src, dst, _attb(att1))
    st = _stats(h01, h23)
    xl, xr = _bn_act_proj(h01, h23, st, g1.reshape(1, HC), be1.reshape(1, HC),
                          Wl2, Wr2)
    h01, h23 = _conv(xl, xr, src, dst, _attb(att2))
    st = _stats(h01, h23)
    hnh = _bn_act_halves(h01, h23, st, g2.reshape(1, HC), be2.reshape(1, HC))
    psum, pcnt = _pool(hnh, batch)
    return _head(psum, pcnt, fc1_w, gfc.reshape(1, HID), bfc.reshape(1, HID),
                 fc2_w, gemb.reshape(1, EMB), bemb.reshape(1, EMB))


# speedup vs baseline: 1.9491x; 1.9491x over previous
import jax
import jax.numpy as jnp
from jax import lax
from jax.experimental import pallas as pl
from jax.experimental.pallas import tpu as pltpu
from jax.experimental.pallas import tpu_sc as plsc

N = 10000
E = 160000
F_IN = 128
HC = 256
HQ = 64
AW = 80
G = 64
HID = 64
EMB = 256
NC, NS, L = 2, 16, 16

EC = 80
E_TILE = E // NS
N_EC = E_TILE // EC
GRP = EC // L
ROWS_TILE = N // NS
NG = N // L
DIV_ITERS = -(-NG // NS)

RB = 1000
RC = 200
N_RC = N // RC
POOL_ITERS = -(-N_RC // NS)


def _sc_mesh():
    return plsc.VectorSubcoreMesh(
        core_axis_name="c", subcore_axis_name="s",
        num_cores=NC, num_subcores=NS)


_SC_PARAMS = pltpu.CompilerParams(needs_layout_passes=False,
                                  use_tc_tiling_on_sc=False,
                                  disable_bounds_checks=True)


def _conv_body(call, xlh, xrh, src_h, dst_h, attb_h, out_h,
               xl0_v, xr0_v, xl1_v, xr1_v, val0_v, val1_v, attb_v,
               obuf_v, dbuf_v, srcs_v, dsts_v,
               acc_s, g0a, g0b, g1a, g1b):
    c = lax.axis_index("c")
    s = lax.axis_index("s")
    head = 2 * call + c
    iota = lax.iota(jnp.int32, L)
    zf = jnp.zeros((L,), jnp.float32)

    pltpu.sync_copy(attb_h.at[head], attb_v)
    pltpu.sync_copy(src_h.at[s], srcs_v)
    pltpu.sync_copy(dst_h.at[s], dsts_v)
    att4 = [attb_v[pl.ds(q * L, L)] for q in range(HQ // L)]

    def _zero_val(r, carry):
        for q in range(AW // L):
            val0_v[r, pl.ds(q * L, L)] = zf
            val1_v[r, pl.ds(q * L, L)] = zf
        return carry
    lax.fori_loop(0, EC, _zero_val, 0)

    row0 = s * ROWS_TILE

    def _zero_acc(k, carry):
        pltpu.sync_copy(val0_v, acc_s.at[pl.ds(row0 + k * EC, EC)])
        return carry
    lax.fori_loop(0, ROWS_TILE // EC, _zero_acc, 0)
    rem = ROWS_TILE % EC
    pltpu.sync_copy(val0_v.at[pl.ds(0, rem)],
                    acc_s.at[pl.ds(row0 + ROWS_TILE - rem, rem)])

    plsc.subcore_barrier()

    def _issue(k, xl_v, xr_v, sa, sb):
        da = pltpu.async_copy(xlh.at[head].at[srcs_v.at[k]], xl_v, sa)
        db = pltpu.async_copy(xrh.at[head].at[dsts_v.at[k]], xr_v, sb)
        return da, db

    def _compute(k, xl_v, xr_v, val_v):
        def _group(g, gcarry):
            base = g * L
            av = zf
            for j in range(L):
                e = base + j
                dot = zf
                for q in range(HQ // L):
                    a = xl_v[e, pl.ds(q * L, L)]
                    b = xr_v[e, pl.ds(q * L, L)]
                    t = a + b
                    m = jnp.maximum(t, t * 0.2)
                    dot = dot + m * att4[q]
                av = jnp.where(iota == j, jnp.sum(dot), av)
            ae = jnp.exp(av)
            plsc.store_scatter(val_v, [base + iota,
                                       jnp.full((L,), HQ, jnp.int32)], ae)
            for j in range(L):
                e = base + j
                aej = ae[j]
                for q in range(HQ // L):
                    val_v[e, pl.ds(q * L, L)] = xl_v[e, pl.ds(q * L, L)] * aej
            return gcarry
        lax.fori_loop(0, GRP, _group, 0)
        pltpu.sync_copy(val_v, acc_s.at[dsts_v.at[k]], add=True)

    _issue(0, xl0_v, xr0_v, g0a, g0b)

    def _pair(t, carry):
        k0 = 2 * t
        k1 = 2 * t + 1
        _issue(k1, xl1_v, xr1_v, g1a, g1b)
        pltpu.make_async_copy(xlh.at[head].at[srcs_v.at[k0]], xl0_v, g0a).wait()
        pltpu.make_async_copy(xrh.at[head].at[dsts_v.at[k0]], xr0_v, g0b).wait()
        _compute(k0, xl0_v, xr0_v, val0_v)
        _issue(k1 + 1, xl0_v, xr0_v, g0a, g0b)
        pltpu.make_async_copy(xlh.at[head].at[srcs_v.at[k1]], xl1_v, g1a).wait()
        pltpu.make_async_copy(xrh.at[head].at[dsts_v.at[k1]], xr1_v, g1b).wait()
        _compute(k1, xl1_v, xr1_v, val1_v)
        return carry
    lax.fori_loop(0, (N_EC - 1) // 2, _pair, 0)

    kt = N_EC - 1
    pltpu.make_async_copy(xlh.at[head].at[srcs_v.at[kt]], xl0_v, g0a).wait()
    pltpu.make_async_copy(xrh.at[head].at[dsts_v.at[kt]], xr0_v, g0b).wait()
    _compute(kt, xl0_v, xr0_v, val0_v)

    plsc.subcore_barrier()

    def _div(t, carry):
        gi = s + t * NS

        @pl.when(gi < NG)
        def _():
            rb = gi * L
            pltpu.sync_copy(acc_s.at[pl.ds(rb, L)], dbuf_v)
            for j in range(L):
                dv = dbuf_v[j, pl.ds(HQ, L)]
                inv = (1.0 / (dv + 1e-16))[0]
                for q in range(HQ // L):
                    obuf_v[j, pl.ds(q * L, L)] = (
                        dbuf_v[j, pl.ds(q * L, L)] * inv)
            pltpu.sync_copy(obuf_v, out_h.at[c, pl.ds(rb, L)])
        return carry
    lax.fori_loop(0, DIV_ITERS, _div, 0)


def _conv_heads(call, xlq, xrq, src, dst, attb):
    def body(*refs):
        _conv_body(call, *refs)
    k = pl.kernel(
        body,
        out_type=jax.ShapeDtypeStruct((NC, N, HQ), jnp.float32),
        mesh=_sc_mesh(), compiler_params=_SC_PARAMS,
        scratch_types=[
            pltpu.VMEM((EC, HQ), jnp.float32),
            pltpu.VMEM((EC, HQ), jnp.float32),
            pltpu.VMEM((EC, HQ), jnp.float32),
            pltpu.VMEM((EC, HQ), jnp.float32),
            pltpu.VMEM((EC, AW), jnp.float32),
            pltpu.VMEM((EC, AW), jnp.float32),
            pltpu.VMEM((HQ,), jnp.float32),
            pltpu.VMEM((L, HQ), jnp.float32),
            pltpu.VMEM((L, AW), jnp.float32),
            pltpu.VMEM((N_EC, EC), jnp.int32),
            pltpu.VMEM((N_EC, EC), jnp.int32),
            pltpu.VMEM_SHARED((N, AW), jnp.float32),
            pltpu.SemaphoreType.DMA,
            pltpu.SemaphoreType.DMA,
            pltpu.SemaphoreType.DMA,
            pltpu.SemaphoreType.DMA,
        ])
    return k(xlq, xrq, src, dst, attb)


def _pool_body(h_h, batch_h, psum_h, pcnt_h,
               rows_v, ones_v, zb_v, bidx_v, acc_s, cnt_s):
    c = lax.axis_index("c")
    s = lax.axis_index("s")
    onesf = jnp.ones((L,), jnp.float32)
    zf = jnp.zeros((L,), jnp.float32)

    def _fill_ones(r, carry):
        ones_v[r, :] = onesf
        return carry
    lax.fori_loop(0, RC, _fill_ones, 0)

    def _fill_z(r, carry):
        zb_v[r, :] = zf
        for q in range(128 // L):
            rows_v[r, pl.ds(q * L, L)] = zf
        return carry
    lax.fori_loop(0, G, _fill_z, 0)

    @pl.when(s == 0)
    def _():
        pltpu.sync_copy(rows_v.at[pl.ds(0, G)], acc_s)
        pltpu.sync_copy(zb_v, cnt_s)

    plsc.subcore_barrier()

    def _chunk(k, carry):
        cid = s + NS * k

        @pl.when(cid < N_RC)
        def _():
            base = cid * RC
            pltpu.sync_copy(h_h.at[c].at[pl.ds(base, RC)], rows_v)
            pltpu.sync_copy(batch_h.at[pl.ds(base, RC)], bidx_v)
            pltpu.sync_copy(rows_v, acc_s.at[bidx_v], add=True)
            pltpu.sync_copy(ones_v, cnt_s.at[bidx_v], add=True)
        return carry
    lax.fori_loop(0, POOL_ITERS, _chunk, 0)

    plsc.subcore_barrier()

    @pl.when(s == 0)
    def _():
        pltpu.sync_copy(acc_s, psum_h.at[c])
        pltpu.sync_copy(cnt_s, pcnt_h.at[c])


def _pool(hnh, batch):
    k = pl.kernel(
        _pool_body,
        out_type=(jax.ShapeDtypeStruct((NC, G, 128), jnp.float32),
                  jax.ShapeDtypeStruct((NC, G, L), jnp.float32)),
        mesh=_sc_mesh(), compiler_params=_SC_PARAMS,
        scratch_types=[
            pltpu.VMEM((RC, 128), jnp.float32),
            pltpu.VMEM((RC, L), jnp.float32),
            pltpu.VMEM((G, L), jnp.float32),
            pltpu.VMEM((RC,), jnp.int32),
            pltpu.VMEM_SHARED((G, 128), jnp.float32),
            pltpu.VMEM_SHARED((G, L), jnp.float32),
        ])
    return k(hnh, batch)


def _q_specs(grid_rank):
    if grid_rank == 1:
        maps = [lambda i: (0, i, 0), lambda i: (1, i, 0)]
    else:
        maps = [lambda i, j: (0, i, 0), lambda i, j: (1, i, 0)]
    return [pl.BlockSpec((1, RB, HQ), m) for m in maps for _ in (0,)]


def _proj_in(x, Wl, Wr):
    def body(x_ref, wl_ref, wr_ref, xl_ref, xr_ref):
        h = x_ref[...]
        rl = jnp.dot(h, wl_ref[...], preferred_element_type=jnp.float32)
        rr = jnp.dot(h, wr_ref[...], preferred_element_type=jnp.float32)
        for q in range(4):
            xl_ref[q] = rl[:, q * HQ:(q + 1) * HQ]
            xr_ref[q] = rr[:, q * HQ:(q + 1) * HQ]
    return pl.pallas_call(
        body,
        grid=(N // RB,),
        in_specs=[pl.BlockSpec((RB, F_IN), lambda i: (i, 0)),
                  pl.BlockSpec((F_IN, HC), lambda i: (0, 0)),
                  pl.BlockSpec((F_IN, HC), lambda i: (0, 0))],
        out_specs=[pl.BlockSpec((4, RB, HQ), lambda i: (0, i, 0)),
                   pl.BlockSpec((4, RB, HQ), lambda i: (0, i, 0))],
        out_shape=[jax.ShapeDtypeStruct((4, N, HQ), jnp.float32)] * 2,
    )(x, Wl, Wr)


def _quarter_in_specs(grid2):
    if grid2:
        return [pl.BlockSpec((1, RB, HQ), lambda i, j: (0, i, 0)),
                pl.BlockSpec((1, RB, HQ), lambda i, j: (1, i, 0)),
                pl.BlockSpec((1, RB, HQ), lambda i, j: (0, i, 0)),
                pl.BlockSpec((1, RB, HQ), lambda i, j: (1, i, 0))]
    return [pl.BlockSpec((1, RB, HQ), lambda i: (0, i, 0)),
            pl.BlockSpec((1, RB, HQ), lambda i: (1, i, 0)),
            pl.BlockSpec((1, RB, HQ), lambda i: (0, i, 0)),
            pl.BlockSpec((1, RB, HQ), lambda i: (1, i, 0))]


def _cat4(a0, a1, b0, b1):
    return jnp.concatenate([a0[0], a1[0], b0[0], b1[0]], axis=1)


def _stats(h01, h23):
    def body(a0, a1, b0, b1, o_ref):
        i = pl.program_id(0)
        hb = _cat4(a0, a1, b0, b1)
        blk = jnp.concatenate(
            [jnp.sum(hb, 0)[None], jnp.sum(hb * hb, 0)[None],
             jnp.zeros((6, HC), jnp.float32)], 0)

        @pl.when(i == 0)
        def _():
            o_ref[...] = jnp.zeros_like(o_ref)
        o_ref[...] += blk
    return pl.pallas_call(
        body,
        grid=(N // RB,),
        in_specs=_quarter_in_specs(False),
        out_specs=pl.BlockSpec((8, HC), lambda i: (0, 0)),
        out_shape=jax.ShapeDtypeStruct((8, HC), jnp.float32),
    )(h01, h01, h23, h23)


def _bn_from_stats(hb, st, g, be):
    mu = st[0] / N
    var = st[1] / N - mu * mu
    xn = g * (hb - mu) * lax.rsqrt(var + 1e-5) + be
    return jnp.maximum(xn, xn * 0.01)


def _bn_act_proj(h01, h23, st, g2d, be2d, Wl, Wr):
    def body(a0, a1, b0, b1, st_ref, g_ref, be_ref, wl_ref, wr_ref,
             xl_ref, xr_ref):
        hb = _cat4(a0, a1, b0, b1)
        hn = _bn_from_stats(hb, st_ref[...], g_ref[0], be_ref[0])
        rl = jnp.dot(hn, wl_ref[...], preferred_element_type=jnp.float32)
        rr = jnp.dot(hn, wr_ref[...], preferred_element_type=jnp.float32)
        for q in range(4):
            xl_ref[q] = rl[:, q * HQ:(q + 1) * HQ]
            xr_ref[q] = rr[:, q * HQ:(q + 1) * HQ]
    return pl.pallas_call(
        body,
        grid=(N // RB,),
        in_specs=_quarter_in_specs(False) + [
            pl.BlockSpec((8, HC), lambda i: (0, 0)),
            pl.BlockSpec((1, HC), lambda i: (0, 0)),
            pl.BlockSpec((1, HC), lambda i: (0, 0)),
            pl.BlockSpec((HC, HC), lambda i: (0, 0)),
            pl.BlockSpec((HC, HC), lambda i: (0, 0))],
        out_specs=[pl.BlockSpec((4, RB, HQ), lambda i: (0, i, 0)),
                   pl.BlockSpec((4, RB, HQ), lambda i: (0, i, 0))],
        out_shape=[jax.ShapeDtypeStruct((4, N, HQ), jnp.float32)] * 2,
    )(h01, h01, h23, h23, st, g2d, be2d, Wl, Wr)


def _bn_act_halves(h01, h23, st, g2d, be2d):
    def body(a0, a1, b0, b1, st_ref, g_ref, be_ref, o_ref):
        hb = _cat4(a0, a1, b0, b1)
        hn = _bn_from_stats(hb, st_ref[...], g_ref[0], be_ref[0])
        o_ref[0] = hn[:, :128]
        o_ref[1] = hn[:, 128:]
    return pl.pallas_call(
        body,
        grid=(N // RB,),
        in_specs=_quarter_in_specs(False) + [
            pl.BlockSpec((8, HC), lambda i: (0, 0)),
            pl.BlockSpec((1, HC), lambda i: (0, 0)),
            pl.BlockSpec((1, HC), lambda i: (0, 0))],
        out_specs=pl.BlockSpec((2, RB, 128), lambda i: (0, i, 0)),
        out_shape=jax.ShapeDtypeStruct((2, N, 128), jnp.float32),
    )(h01, h01, h23, h23, st, g2d, be2d)


def _head(psum, pcnt, fc1_w, gfc, bfc, fc2_w, gemb, bemb):
    def body(ps_ref, pc_ref, w1_ref, g1_ref, b1_ref, w2_ref, g2_ref, b2_ref,
             o_ref):
        ps = ps_ref[...]
        pc = pc_ref[...]
        sums = jnp.concatenate([ps[0], ps[1]], axis=1)
        cnt = pc[0, :, 0:1]
        hg = sums / jnp.maximum(cnt, 1.0)
        z = jnp.dot(hg, w1_ref[...], preferred_element_type=jnp.float32)
        mu = jnp.mean(z, 0)
        var = jnp.mean(z * z, 0) - mu * mu
        z = g1_ref[0] * (z - mu) * lax.rsqrt(var + 1e-5) + b1_ref[0]
        z = jnp.maximum(z, z * 0.01)
        z2 = jnp.dot(z, w2_ref[...], preferred_element_type=jnp.float32)
        mu2 = jnp.mean(z2, 0)
        var2 = jnp.mean(z2 * z2, 0) - mu2 * mu2
        z2 = g2_ref[0] * (z2 - mu2) * lax.rsqrt(var2 + 1e-5) + b2_ref[0]
        nrm = jnp.sqrt(jnp.sum(z2 * z2, 1, keepdims=True))
        o_ref[...] = z2 / jnp.maximum(nrm, 1e-12)
    return pl.pallas_call(
        body,
        out_shape=jax.ShapeDtypeStruct((G, EMB), jnp.float32),
    )(psum, pcnt, fc1_w, gfc, bfc, fc2_w, gemb, bemb)


def _attb(att):
    return att


def _conv(xlq, xrq, src3, dst3, attb):
    h01 = _conv_heads(0, xlq, xrq, src3, dst3, attb)
    h23 = _conv_heads(1, xlq, xrq, src3, dst3, attb)
    return h01, h23


def kernel(x, edge_index, batch,
           Wl0, Wr0, att0, b0, g0, be0,
           Wl1, Wr1, att1, b1, g1, be1,
           Wl2, Wr2, att2, b2, g2, be2,
           fc1_w, fc1_b, gfc, bfc, fc2_w, fc2_b, gemb, bemb):
    src = edge_index[0].reshape(NS, N_EC, EC)
    dst = edge_index[1].reshape(NS, N_EC, EC)
    xl, xr = _proj_in(x, Wl0, Wr0)
    h01, h23 = _conv(xl, xr, src, dst, _attb(att0))
    st = _stats(h01, h23)
    xl, xr = _bn_act_proj(h01, h23, st, g0.reshape(1, HC), be0.reshape(1, HC),
                          Wl1, Wr1)
    h01, h23 = _conv(xl, xr, src, dst, _attb(att1))
    st = _stats(h01, h23)
    xl, xr = _bn_act_proj(h01, h23, st, g1.reshape(1, HC), be1.reshape(1, HC),
                          Wl2, Wr2)
    h01, h23 = _conv(xl, xr, src, dst, _attb(att2))
    st = _stats(h01, h23)
    hnh = _bn_act_halves(h01, h23, st, g2.reshape(1, HC), be2.reshape(1, HC))
    psum, pcnt = _pool(hnh, batch)
    return _head(psum, pcnt, fc1_w, gfc.reshape(1, HID), bfc.reshape(1, HID),
                 fc2_w, gemb.reshape(1, EMB), bemb.reshape(1, EMB))
